# baseline (speedup 1.0000x reference)
import functools

import jax
import jax.numpy as jnp
from jax import lax
from jax.experimental import pallas as pl
from jax.experimental.pallas import tpu as pltpu
from jax.experimental.pallas import tpu_sc as plsc

B, C_IN, N = 2, 192, 1024
D, K = 64, 512
_DP = 128

_NC, _NS, _L = 2, 16, 16
_NW = _NC * _NS
_TOK = B * N
_TPW = _TOK // _NW


def _tc_body(z_ref, w_ref, emb_ref, idx_ref, embp_ref):
    zb = z_ref[0]
    w = w_ref[...]
    emb = emb_ref[...]
    hi = lax.Precision.HIGHEST
    ze = jnp.dot(w.astype(jnp.bfloat16), zb.astype(jnp.bfloat16),
                 preferred_element_type=jnp.float32)
    scores = jnp.dot(emb, ze, preferred_element_type=jnp.float32,
                     precision=hi)
    esq = jnp.sum(emb * emb, axis=1, keepdims=True)
    dist = esq - 2.0 * scores

    iota = lax.broadcasted_iota(jnp.int32, (K, N), 0)
    m1 = jnp.min(dist, axis=0, keepdims=True)
    i1 = jnp.min(jnp.where(dist == m1, iota, K), axis=0, keepdims=True)
    dist2 = jnp.where(iota == i1, jnp.float32(jnp.inf), dist)
    m2 = jnp.min(dist2, axis=0, keepdims=True)
    i2 = jnp.min(jnp.where(dist2 == m2, iota, K), axis=0, keepdims=True)

    oh1 = (iota == i1).astype(jnp.float32)
    oh2 = (iota == i2).astype(jnp.float32)
    dn = (((0,), (0,)), ((), ()))
    e1 = lax.dot_general(emb, oh1, dn, precision=hi,
                         preferred_element_type=jnp.float32)
    e2 = lax.dot_general(emb, oh2, dn, precision=hi,
                         preferred_element_type=jnp.float32)
    d1 = jnp.sum((ze - e1) ** 2, axis=0, keepdims=True)
    d2 = jnp.sum((ze - e2) ** 2, axis=0, keepdims=True)
    pick2 = (d2 < d1) | ((d2 == d1) & (i2 < i1))
    idx_ref[0] = jnp.where(pick2, i2, i1)

    @pl.when(pl.program_id(0) == 0)
    def _():
        embp_ref[:, :D] = emb


_tc_call = pl.pallas_call(
    _tc_body,
    grid=(B,),
    in_specs=[
        pl.BlockSpec((1, C_IN, N), lambda b: (b, 0, 0)),
        pl.BlockSpec((D, C_IN), lambda b: (0, 0)),
        pl.BlockSpec((K, D), lambda b: (0, 0)),
    ],
    out_specs=[
        pl.BlockSpec((1, 1, N), lambda b: (b, 0, 0)),
        pl.BlockSpec((K, _DP), lambda b: (0, 0)),
    ],
    out_shape=[
        jax.ShapeDtypeStruct((B, 1, N), jnp.int32),
        jax.ShapeDtypeStruct((K, _DP), jnp.float32),
    ],
)


@functools.cache
def _make_sc_gather():
    mesh = plsc.VectorSubcoreMesh(core_axis_name="c", subcore_axis_name="s")

    tblk = 128

    @functools.partial(
        pl.kernel,
        mesh=mesh,
        compiler_params=pltpu.CompilerParams(needs_layout_passes=False),
        out_type=jax.ShapeDtypeStruct((B, D, N), jnp.float32),
        scratch_types=[
            pltpu.VMEM((tblk,), jnp.int32),
            pltpu.VMEM((tblk, _DP), jnp.float32),
            pltpu.VMEM((D // 2, tblk), jnp.float32),
            pltpu.SemaphoreType.DMA,
        ],
    )
    def _sc_gather(emb_hbm, idx_hbm, out_hbm, idx_v, rows_v, outt_v, sem):
        wid = lax.axis_index("s") * _NC + lax.axis_index("c")
        b = wid // (_NW // B)
        r = wid % (_NW // B)
        dh = r // (N // tblk)
        tb = r % (N // tblk)
        base = b * N + tb * tblk
        pltpu.sync_copy(idx_hbm.at[pl.ds(base, tblk)], idx_v)
        pltpu.async_copy(emb_hbm.at[idx_v], rows_v, sem).wait()
        lane = lax.broadcasted_iota(jnp.int32, (_L,), 0)
        dbase = dh * (D // 2)
        for t in range(tblk):
            trow = jnp.full((_L,), t, jnp.int32)
            for g in range(D // 2 // _L):
                v = plsc.load_gather(rows_v, [trow, lane + (dbase + g * _L)])
                plsc.store_scatter(outt_v, [lane + g * _L, trow], v)
        pltpu.sync_copy(
            outt_v, out_hbm.at[b, pl.ds(dh * (D // 2), D // 2),
                               pl.ds(tb * tblk, tblk)])

    return _sc_gather


def kernel(z, W, emb):
    idx, emb_p = _tc_call(z, W, emb)
    return _make_sc_gather()(emb_p, idx.reshape(_TOK))
